# baseline (speedup 1.0000x reference)
import functools

import jax
import jax.numpy as jnp
from jax import lax
from jax.experimental import pallas as pl
from jax.experimental.pallas import tpu as pltpu
from jax.experimental.pallas import tpu_sc as plsc

NUM_CORES = 2
NUM_SUBCORES = 16
NUM_WORKERS = NUM_CORES * NUM_SUBCORES

B = 16384 * 50
D = 32
PER_W = B // NUM_WORKERS
CHUNK = 128
STEPS = PER_W // CHUNK


def _emb_kernel(x_hbm, table_hbm, out_hbm, idx_v, rows_v, sem):
  wid = lax.axis_index("s") * NUM_CORES + lax.axis_index("c")
  base = wid * PER_W

  def body(i, _):
    off = base + i * CHUNK
    pltpu.sync_copy(x_hbm.at[pl.ds(off, CHUNK)], idx_v)
    pltpu.async_copy(table_hbm.at[idx_v], rows_v, sem).wait()
    pltpu.sync_copy(rows_v, out_hbm.at[pl.ds(off, CHUNK)])
    return 0

  lax.fori_loop(0, STEPS, body, 0)


@jax.jit
def _emb(x_flat, table):
  mesh = plsc.VectorSubcoreMesh(
      core_axis_name="c", subcore_axis_name="s",
      num_cores=NUM_CORES, num_subcores=NUM_SUBCORES)
  f = pl.kernel(
      _emb_kernel,
      out_type=jax.ShapeDtypeStruct((B, D), jnp.float32),
      mesh=mesh,
      scratch_types=[
          pltpu.VMEM((CHUNK,), jnp.int32),
          pltpu.VMEM((CHUNK, D), jnp.float32),
          pltpu.SemaphoreType.DMA,
      ],
      compiler_params=pltpu.CompilerParams(use_tc_tiling_on_sc=False),
  )
  return f(x_flat, table)


def kernel(x, table):
  x_flat = x.reshape(-1).astype(jnp.int32)
  out = _emb(x_flat, table)
  return out.reshape(x.shape + (D,))


# speedup vs baseline: 1.1354x; 1.1354x over previous
import jax
import jax.numpy as jnp
from jax import lax
from jax.experimental import pallas as pl
from jax.experimental.pallas import tpu as pltpu
from jax.experimental.pallas import tpu_sc as plsc

NUM_CORES = 2
NUM_SUBCORES = 16
NUM_WORKERS = NUM_CORES * NUM_SUBCORES

B = 16384 * 50
D = 32
PER_W = B // NUM_WORKERS
GCHUNK = 128
K = 8
GROUP = K * GCHUNK
NGROUPS = PER_W // GROUP


def _emb_kernel(x_hbm, table_hbm, out_hbm, idx_v, rows_v, gsem, osem):
  wid = lax.axis_index("s") * NUM_CORES + lax.axis_index("c")
  base = wid * PER_W
  pltpu.sync_copy(x_hbm.at[pl.ds(base, PER_W)], idx_v)

  def fire(g, slot):
    for j in range(K):
      pltpu.async_copy(
          table_hbm.at[idx_v.at[pl.ds(g * GROUP + j * GCHUNK, GCHUNK)]],
          rows_v.at[slot, pl.ds(j * GCHUNK, GCHUNK)],
          gsem)

  def drain_gathers(slot):
    pltpu.make_async_copy(
        table_hbm.at[pl.ds(0, GROUP)], rows_v.at[slot], gsem).wait()

  def out_copy(g, slot):
    pltpu.async_copy(rows_v.at[slot],
                     out_hbm.at[pl.ds(base + g * GROUP, GROUP)], osem)

  def drain_out(g, slot):
    pltpu.make_async_copy(
        rows_v.at[slot], out_hbm.at[pl.ds(base + g * GROUP, GROUP)],
        osem).wait()

  fire(0, 0)

  def body(g, _):
    slot = g % 2
    drain_gathers(slot)

    @pl.when(g + 1 < NGROUPS)
    def _():
      @pl.when(g >= 1)
      def _():
        drain_out(g - 1, 1 - slot)
      fire(g + 1, 1 - slot)

    out_copy(g, slot)
    return 0

  lax.fori_loop(0, NGROUPS, body, 0)
  drain_out(NGROUPS - 2, 1)
  drain_out(NGROUPS - 1, 0)


@jax.jit
def _emb(x_flat, table):
  mesh = plsc.VectorSubcoreMesh(
      core_axis_name="c", subcore_axis_name="s",
      num_cores=NUM_CORES, num_subcores=NUM_SUBCORES)
  f = pl.kernel(
      _emb_kernel,
      out_type=jax.ShapeDtypeStruct((B, D), jnp.float32),
      mesh=mesh,
      scratch_types=[
          pltpu.VMEM((PER_W,), jnp.int32),
          pltpu.VMEM((2, GROUP, D), jnp.float32),
          pltpu.SemaphoreType.DMA,
          pltpu.SemaphoreType.DMA,
      ],
      compiler_params=pltpu.CompilerParams(use_tc_tiling_on_sc=False),
  )
  return f(x_flat, table)


def kernel(x, table):
  x_flat = x.reshape(-1).astype(jnp.int32)
  out = _emb(x_flat, table)
  return out.reshape(x.shape + (D,))


# speedup vs baseline: 1.1392x; 1.0034x over previous
import jax
import jax.numpy as jnp
from jax import lax
from jax.experimental import pallas as pl
from jax.experimental.pallas import tpu as pltpu
from jax.experimental.pallas import tpu_sc as plsc

NUM_CORES = 2
NUM_SUBCORES = 16
NUM_WORKERS = NUM_CORES * NUM_SUBCORES

B = 16384 * 50
D = 32
PER_W = B // NUM_WORKERS
GCHUNK = 128
K = 8
GROUP = K * GCHUNK
NGROUPS = PER_W // GROUP
NBUF = 3


def _emb_kernel(x_hbm, table_hbm, out_hbm, idx_v, rows_v, gsem, osem):
  wid = lax.axis_index("s") * NUM_CORES + lax.axis_index("c")
  base = wid * PER_W
  pltpu.sync_copy(x_hbm.at[pl.ds(base, PER_W)], idx_v)

  def fire(g, slot):
    for j in range(K):
      pltpu.async_copy(
          table_hbm.at[idx_v.at[pl.ds(g * GROUP + j * GCHUNK, GCHUNK)]],
          rows_v.at[slot, pl.ds(j * GCHUNK, GCHUNK)],
          gsem.at[slot])

  def drain_gathers(slot):
    pltpu.make_async_copy(
        table_hbm.at[pl.ds(0, GROUP)], rows_v.at[slot], gsem.at[slot]).wait()

  def out_copy(g, slot):
    pltpu.async_copy(rows_v.at[slot],
                     out_hbm.at[pl.ds(base + g * GROUP, GROUP)], osem.at[slot])

  def drain_out(g, slot):
    pltpu.make_async_copy(
        rows_v.at[slot], out_hbm.at[pl.ds(base + g * GROUP, GROUP)],
        osem.at[slot]).wait()

  fire(0, 0)
  fire(1, 1)

  def body(g, _):
    slot = g % NBUF

    @pl.when(g + 2 < NGROUPS)
    def _():
      nslot = (g + 2) % NBUF

      @pl.when(g >= 1)
      def _():
        drain_out(g - 1, nslot)
      fire(g + 2, nslot)

    drain_gathers(slot)
    out_copy(g, slot)
    return 0

  lax.fori_loop(0, NGROUPS, body, 0)
  drain_out(NGROUPS - 3, (NGROUPS - 3) % NBUF)
  drain_out(NGROUPS - 2, (NGROUPS - 2) % NBUF)
  drain_out(NGROUPS - 1, (NGROUPS - 1) % NBUF)


@jax.jit
def _emb(x_flat, table):
  mesh = plsc.VectorSubcoreMesh(
      core_axis_name="c", subcore_axis_name="s",
      num_cores=NUM_CORES, num_subcores=NUM_SUBCORES)
  f = pl.kernel(
      _emb_kernel,
      out_type=jax.ShapeDtypeStruct((B, D), jnp.float32),
      mesh=mesh,
      scratch_types=[
          pltpu.VMEM((PER_W,), jnp.int32),
          pltpu.VMEM((NBUF, GROUP, D), jnp.float32),
          pltpu.SemaphoreType.DMA((NBUF,)),
          pltpu.SemaphoreType.DMA((NBUF,)),
      ],
      compiler_params=pltpu.CompilerParams(use_tc_tiling_on_sc=False),
  )
  return f(x_flat, table)


def kernel(x, table):
  x_flat = x.reshape(-1).astype(jnp.int32)
  out = _emb(x_flat, table)
  return out.reshape(x.shape + (D,))
